# Initial kernel scaffold; baseline (speedup 1.0000x reference)
import jax
import jax.numpy as jnp
from jax.experimental import pallas as pl


def kernel(x, mask_pattern):
    raise NotImplementedError("write your pallas kernel here")


# speedup vs baseline: 3.2942x; 3.2942x over previous
import functools

import jax
import jax.numpy as jnp
from jax.experimental import pallas as pl


def _tile_kernel(x_ref, mpT_ref, mp_ref, sparse_ref, mask_ref):
    x = x_ref[...]
    tm, tn = x.shape
    nb = (tm // 4) * (tn // 4)
    blocks = (
        x.reshape(tm // 4, 4, tn // 4, 4)
        .transpose(0, 2, 1, 3)
        .reshape(nb, 16)
    )
    scores = jax.lax.dot(
        jnp.abs(blocks), mpT_ref[...], preferred_element_type=jnp.float32
    )
    mx = jnp.max(scores, axis=1, keepdims=True)
    idx = jax.lax.broadcasted_iota(jnp.int32, scores.shape, 1)
    best = jnp.min(jnp.where(scores == mx, idx, 90), axis=1, keepdims=True)
    onehot = (idx == best).astype(jnp.float32)
    maskv = jax.lax.dot(
        onehot, mp_ref[...], preferred_element_type=jnp.float32
    )
    sb = blocks * maskv
    sparse_ref[...] = (
        sb.reshape(tm // 4, tn // 4, 4, 4).transpose(0, 2, 1, 3).reshape(tm, tn)
    )
    maskf = (
        maskv.reshape(tm // 4, tn // 4, 4, 4).transpose(0, 2, 1, 3).reshape(tm, tn)
    )
    mask_ref[...] = maskf > 0.5


@functools.partial(jax.jit, static_argnames=("tm",))
def _run(x, mpT, mp, tm):
    m, k = x.shape
    grid = (m // tm,)
    sparse, mask = pl.pallas_call(
        _tile_kernel,
        grid=grid,
        in_specs=[
            pl.BlockSpec((tm, k), lambda i: (i, 0)),
            pl.BlockSpec((16, 90), lambda i: (0, 0)),
            pl.BlockSpec((90, 16), lambda i: (0, 0)),
        ],
        out_specs=[
            pl.BlockSpec((tm, k), lambda i: (i, 0)),
            pl.BlockSpec((tm, k), lambda i: (i, 0)),
        ],
        out_shape=[
            jax.ShapeDtypeStruct((m, k), jnp.float32),
            jax.ShapeDtypeStruct((m, k), jnp.bool_),
        ],
    )(x, mpT, mp)
    return sparse, mask


def kernel(x, mask_pattern):
    mp = mask_pattern.astype(jnp.float32)
    return _run(x, mp.T, mp, 32)
